# Initial kernel scaffold; baseline (speedup 1.0000x reference)
import jax
import jax.numpy as jnp
from jax.experimental import pallas as pl


def kernel(x, edge_index, W1_l, b1, W1_r, W2_l, b2, W2_r):
    raise NotImplementedError("write your pallas kernel here")


# speedup vs baseline: 6.4609x; 6.4609x over previous
import jax
import jax.numpy as jnp
from jax import lax
from jax.experimental import pallas as pl
from jax.experimental.pallas import tpu as pltpu
from jax.experimental.pallas import tpu_sc as plsc

N_NODES = 10000
N_EDGES = 320000
D_IN, D_HID, D_OUT = 128, 64, 32

NC, NS = 2, 16
NW = NC * NS
EB = 128
KB = (-(-N_EDGES // (NW * EB)) + 7) // 8 * 8
E_PAD = NW * KB * EB
RPT = 640
N_ACC = NS * RPT

_ROWS = 1000
_GRID = N_NODES // _ROWS


def _lin1_body(x_ref, wl_ref, wr_ref, b_ref, y_ref, r_ref):
    xb = x_ref[...]
    dn = (((1,), (1,)), ((), ()))
    y_ref[...] = lax.dot_general(xb, wl_ref[...], dn,
                                 preferred_element_type=jnp.float32)
    r_ref[...] = lax.dot_general(xb, wr_ref[...], dn,
                                 preferred_element_type=jnp.float32) + b_ref[...]


_lin1 = pl.pallas_call(
    _lin1_body,
    grid=(_GRID,),
    in_specs=[
        pl.BlockSpec((_ROWS, D_IN), lambda i: (i, 0)),
        pl.BlockSpec((D_HID, D_IN), lambda i: (0, 0)),
        pl.BlockSpec((D_HID, D_IN), lambda i: (0, 0)),
        pl.BlockSpec((1, D_HID), lambda i: (0, 0)),
    ],
    out_specs=[pl.BlockSpec((_ROWS, D_HID), lambda i: (i, 0))] * 2,
    out_shape=[jax.ShapeDtypeStruct((N_NODES, D_HID), jnp.float32)] * 2,
)


def _mid_body(sa_ref, sb_ref, ca_ref, cb_ref, r1_ref, wl_ref, wr_ref, b_ref,
              y_ref, r_ref):
    c = ca_ref[...] + cb_ref[...]
    inv = 1.0 / jnp.maximum(c[:, 0:1], 1.0)
    h = jnp.maximum((sa_ref[...] + sb_ref[...]) * inv + r1_ref[...], 0.0)
    dn = (((1,), (1,)), ((), ()))
    y_ref[...] = lax.dot_general(h, wl_ref[...], dn,
                                 preferred_element_type=jnp.float32)
    r_ref[...] = lax.dot_general(h, wr_ref[...], dn,
                                 preferred_element_type=jnp.float32) + b_ref[...]


_mid = pl.pallas_call(
    _mid_body,
    grid=(_GRID,),
    in_specs=[
        pl.BlockSpec((_ROWS, D_HID), lambda i: (i, 0)),
        pl.BlockSpec((_ROWS, D_HID), lambda i: (i, 0)),
        pl.BlockSpec((_ROWS, 16), lambda i: (i, 0)),
        pl.BlockSpec((_ROWS, 16), lambda i: (i, 0)),
        pl.BlockSpec((_ROWS, D_HID), lambda i: (i, 0)),
        pl.BlockSpec((D_OUT, D_HID), lambda i: (0, 0)),
        pl.BlockSpec((D_OUT, D_HID), lambda i: (0, 0)),
        pl.BlockSpec((1, D_OUT), lambda i: (0, 0)),
    ],
    out_specs=[pl.BlockSpec((_ROWS, D_OUT), lambda i: (i, 0))] * 2,
    out_shape=[jax.ShapeDtypeStruct((N_NODES, D_OUT), jnp.float32)] * 2,
)


def _out_body(sa_ref, sb_ref, ca_ref, cb_ref, r2_ref, o_ref):
    c = ca_ref[...] + cb_ref[...]
    inv = 1.0 / jnp.maximum(c[:, 0:1], 1.0)
    o_ref[...] = (sa_ref[...] + sb_ref[...]) * inv + r2_ref[...]


_out = pl.pallas_call(
    _out_body,
    grid=(_GRID,),
    in_specs=[
        pl.BlockSpec((_ROWS, D_OUT), lambda i: (i, 0)),
        pl.BlockSpec((_ROWS, D_OUT), lambda i: (i, 0)),
        pl.BlockSpec((_ROWS, 16), lambda i: (i, 0)),
        pl.BlockSpec((_ROWS, 16), lambda i: (i, 0)),
        pl.BlockSpec((_ROWS, D_OUT), lambda i: (i, 0)),
    ],
    out_specs=pl.BlockSpec((_ROWS, D_OUT), lambda i: (i, 0)),
    out_shape=jax.ShapeDtypeStruct((N_NODES, D_OUT), jnp.float32),
)


_SC_PARAMS = pltpu.CompilerParams(use_tc_tiling_on_sc=False)


def _make_agg(D):
    mesh = plsc.VectorSubcoreMesh(core_axis_name="c", subcore_axis_name="s")
    out_type = [jax.ShapeDtypeStruct((NC, N_ACC, D), jnp.float32)]
    scratch = [
        pltpu.VMEM((KB, EB), jnp.int32),
        pltpu.VMEM((KB, EB), jnp.int32),
        pltpu.VMEM((EB, D), jnp.float32),
        pltpu.VMEM((RPT, D), jnp.float32),
        pltpu.VMEM_SHARED((N_ACC, D), jnp.float32),
        pltpu.SemaphoreType.DMA,
    ]

    def body(y_hbm, src_hbm, dst_hbm, z_hbm, acc_out,
             src_v, dst_v, rows_v, wb_v, acc_sh, sem):
        cid = lax.axis_index("c")
        sid = lax.axis_index("s")
        wid = sid * NC + cid
        pltpu.sync_copy(z_hbm, wb_v)
        pltpu.sync_copy(wb_v, acc_sh.at[pl.ds(sid * RPT, RPT)])
        pltpu.sync_copy(src_hbm.at[pl.ds(wid * KB, KB)], src_v)
        pltpu.sync_copy(dst_hbm.at[pl.ds(wid * KB, KB)], dst_v)
        plsc.subcore_barrier()

        def step(j, carry):
            pltpu.async_copy(y_hbm.at[src_v.at[j]], rows_v, sem).wait()
            pltpu.sync_copy(rows_v, acc_sh.at[dst_v.at[j]], add=True)
            return carry

        lax.fori_loop(0, KB, step, 0)
        plsc.subcore_barrier()
        pltpu.sync_copy(acc_sh.at[pl.ds(sid * RPT, RPT)], wb_v)
        pltpu.sync_copy(wb_v, acc_out.at[cid, pl.ds(sid * RPT, RPT)])

    return pl.kernel(body, out_type=out_type, mesh=mesh,
                     scratch_types=scratch, compiler_params=_SC_PARAMS)


def _counts_body(dst_hbm, z16_hbm, ones_hbm, cnt_out,
                 dst_v, ones_v, wb16_v, cnt_sh):
    cid = lax.axis_index("c")
    sid = lax.axis_index("s")
    wid = sid * NC + cid
    pltpu.sync_copy(z16_hbm, wb16_v)
    pltpu.sync_copy(wb16_v, cnt_sh.at[pl.ds(sid * RPT, RPT)])
    pltpu.sync_copy(ones_hbm, ones_v)
    pltpu.sync_copy(dst_hbm.at[pl.ds(wid * KB, KB)], dst_v)
    plsc.subcore_barrier()

    def step(j, carry):
        pltpu.sync_copy(ones_v, cnt_sh.at[dst_v.at[j]], add=True)
        return carry

    lax.fori_loop(0, KB, step, 0)
    plsc.subcore_barrier()
    pltpu.sync_copy(cnt_sh.at[pl.ds(sid * RPT, RPT)], wb16_v)
    pltpu.sync_copy(wb16_v, cnt_out.at[cid, pl.ds(sid * RPT, RPT)])


_counts = pl.kernel(
    _counts_body,
    out_type=[jax.ShapeDtypeStruct((NC, N_ACC, 16), jnp.float32)],
    mesh=plsc.VectorSubcoreMesh(core_axis_name="c", subcore_axis_name="s"),
    scratch_types=[
        pltpu.VMEM((KB, EB), jnp.int32),
        pltpu.VMEM((EB, 16), jnp.float32),
        pltpu.VMEM((RPT, 16), jnp.float32),
        pltpu.VMEM_SHARED((N_ACC, 16), jnp.float32),
    ],
    compiler_params=_SC_PARAMS,
)


_agg_hid = _make_agg(D_HID)
_agg_out = _make_agg(D_OUT)


def kernel(x, edge_index, W1_l, b1, W1_r, W2_l, b2, W2_r):
    src = edge_index[0].astype(jnp.int32)
    dst = edge_index[1].astype(jnp.int32)
    pad = E_PAD - N_EDGES
    src_p = jnp.concatenate(
        [src, jnp.zeros((pad,), jnp.int32)]).reshape(NW * KB, EB)
    dst_p = jnp.concatenate(
        [dst, jnp.full((pad,), N_NODES, jnp.int32)]).reshape(NW * KB, EB)
    z_hid = jnp.zeros((RPT, D_HID), jnp.float32)
    z_out = jnp.zeros((RPT, D_OUT), jnp.float32)
    z16 = jnp.zeros((RPT, 16), jnp.float32)
    ones = jnp.ones((EB, 16), jnp.float32)

    y1, r1 = _lin1(x, W1_l, W1_r, b1.reshape(1, D_HID))
    (accp,) = _agg_hid(y1, src_p, dst_p, z_hid)
    (cntp,) = _counts(dst_p, z16, ones)
    y2, r2 = _mid(accp[0, :N_NODES], accp[1, :N_NODES],
                  cntp[0, :N_NODES], cntp[1, :N_NODES],
                  r1, W2_l, W2_r, b2.reshape(1, D_OUT))
    (accp2,) = _agg_out(y2, src_p, dst_p, z_out)
    return _out(accp2[0, :N_NODES], accp2[1, :N_NODES],
                cntp[0, :N_NODES], cntp[1, :N_NODES], r2)


# speedup vs baseline: 6.8596x; 1.0617x over previous
import jax
import jax.numpy as jnp
from jax import lax
from jax.experimental import pallas as pl
from jax.experimental.pallas import tpu as pltpu
from jax.experimental.pallas import tpu_sc as plsc

N_NODES = 10000
N_EDGES = 320000
D_IN, D_HID, D_OUT = 128, 64, 32

NC, NS = 2, 16
NW = NC * NS
EB = 128
KB = (-(-N_EDGES // (NW * EB)) + 7) // 8 * 8
E_PAD = NW * KB * EB
RPT = 640
N_ACC = NS * RPT

_ROWS = 1000
_GRID = N_NODES // _ROWS


def _lin1_body(x_ref, wl_ref, wr_ref, b_ref, y_ref, r_ref):
    xb = x_ref[...]
    dn = (((1,), (1,)), ((), ()))
    y_ref[...] = lax.dot_general(xb, wl_ref[...], dn,
                                 preferred_element_type=jnp.float32)
    r_ref[...] = lax.dot_general(xb, wr_ref[...], dn,
                                 preferred_element_type=jnp.float32) + b_ref[...]


_lin1 = pl.pallas_call(
    _lin1_body,
    grid=(_GRID,),
    in_specs=[
        pl.BlockSpec((_ROWS, D_IN), lambda i: (i, 0)),
        pl.BlockSpec((D_HID, D_IN), lambda i: (0, 0)),
        pl.BlockSpec((D_HID, D_IN), lambda i: (0, 0)),
        pl.BlockSpec((1, D_HID), lambda i: (0, 0)),
    ],
    out_specs=[pl.BlockSpec((_ROWS, D_HID), lambda i: (i, 0))] * 2,
    out_shape=[jax.ShapeDtypeStruct((N_NODES, D_HID), jnp.float32)] * 2,
)


def _mid_body(sa_ref, sb_ref, ca_ref, cb_ref, r1_ref, wl_ref, wr_ref, b_ref,
              y_ref, r_ref):
    c = ca_ref[...] + cb_ref[...]
    inv = 1.0 / jnp.maximum(c[:, 0:1], 1.0)
    h = jnp.maximum((sa_ref[...] + sb_ref[...]) * inv + r1_ref[...], 0.0)
    dn = (((1,), (1,)), ((), ()))
    y_ref[...] = lax.dot_general(h, wl_ref[...], dn,
                                 preferred_element_type=jnp.float32)
    r_ref[...] = lax.dot_general(h, wr_ref[...], dn,
                                 preferred_element_type=jnp.float32) + b_ref[...]


_mid = pl.pallas_call(
    _mid_body,
    grid=(_GRID,),
    in_specs=[
        pl.BlockSpec((_ROWS, D_HID), lambda i: (i, 0)),
        pl.BlockSpec((_ROWS, D_HID), lambda i: (i, 0)),
        pl.BlockSpec((_ROWS, 16), lambda i: (i, 0)),
        pl.BlockSpec((_ROWS, 16), lambda i: (i, 0)),
        pl.BlockSpec((_ROWS, D_HID), lambda i: (i, 0)),
        pl.BlockSpec((D_OUT, D_HID), lambda i: (0, 0)),
        pl.BlockSpec((D_OUT, D_HID), lambda i: (0, 0)),
        pl.BlockSpec((1, D_OUT), lambda i: (0, 0)),
    ],
    out_specs=[pl.BlockSpec((_ROWS, D_OUT), lambda i: (i, 0))] * 2,
    out_shape=[jax.ShapeDtypeStruct((N_NODES, D_OUT), jnp.float32)] * 2,
)


def _out_body(sa_ref, sb_ref, ca_ref, cb_ref, r2_ref, o_ref):
    c = ca_ref[...] + cb_ref[...]
    inv = 1.0 / jnp.maximum(c[:, 0:1], 1.0)
    o_ref[...] = (sa_ref[...] + sb_ref[...]) * inv + r2_ref[...]


_out = pl.pallas_call(
    _out_body,
    grid=(_GRID,),
    in_specs=[
        pl.BlockSpec((_ROWS, D_OUT), lambda i: (i, 0)),
        pl.BlockSpec((_ROWS, D_OUT), lambda i: (i, 0)),
        pl.BlockSpec((_ROWS, 16), lambda i: (i, 0)),
        pl.BlockSpec((_ROWS, 16), lambda i: (i, 0)),
        pl.BlockSpec((_ROWS, D_OUT), lambda i: (i, 0)),
    ],
    out_specs=pl.BlockSpec((_ROWS, D_OUT), lambda i: (i, 0)),
    out_shape=jax.ShapeDtypeStruct((N_NODES, D_OUT), jnp.float32),
)


_SC_PARAMS = pltpu.CompilerParams(use_tc_tiling_on_sc=False)


def _make_agg(D):
    mesh = plsc.VectorSubcoreMesh(core_axis_name="c", subcore_axis_name="s")
    out_type = [jax.ShapeDtypeStruct((NC, N_ACC, D), jnp.float32)]
    scratch = [
        pltpu.VMEM((KB, EB), jnp.int32),
        pltpu.VMEM((KB, EB), jnp.int32),
        pltpu.VMEM((2, EB, D), jnp.float32),
        pltpu.VMEM((RPT, D), jnp.float32),
        pltpu.VMEM_SHARED((N_ACC, D), jnp.float32),
        pltpu.SemaphoreType.DMA,
        pltpu.SemaphoreType.DMA,
    ]

    def body(y_hbm, src_hbm, dst_hbm, z_hbm, acc_out,
             src_v, dst_v, rows_v, wb_v, acc_sh, sem_g, sem_s):
        cid = lax.axis_index("c")
        sid = lax.axis_index("s")
        wid = sid * NC + cid
        pltpu.sync_copy(z_hbm, wb_v)
        pltpu.sync_copy(wb_v, acc_sh.at[pl.ds(sid * RPT, RPT)])
        pltpu.sync_copy(src_hbm.at[pl.ds(wid * KB, KB)], src_v)
        pltpu.sync_copy(dst_hbm.at[pl.ds(wid * KB, KB)], dst_v)
        plsc.subcore_barrier()

        pltpu.async_copy(y_hbm.at[src_v.at[0]], rows_v.at[0], sem_g)

        def step(j, carry):
            b = lax.rem(j, 2)
            nb = 1 - b
            pltpu.make_async_copy(
                y_hbm.at[src_v.at[j]], rows_v.at[b], sem_g).wait()

            @pl.when(j >= 1)
            def _():
                pltpu.make_async_copy(
                    rows_v.at[nb], acc_sh.at[dst_v.at[j]], sem_s).wait()

            @pl.when(j + 1 < KB)
            def _():
                pltpu.async_copy(
                    y_hbm.at[src_v.at[j + 1]], rows_v.at[nb], sem_g)

            pltpu.async_copy(
                rows_v.at[b], acc_sh.at[dst_v.at[j]], sem_s, add=True)
            return carry

        lax.fori_loop(0, KB, step, 0)
        pltpu.make_async_copy(
            rows_v.at[(KB - 1) % 2], acc_sh.at[dst_v.at[KB - 1]],
            sem_s).wait()
        plsc.subcore_barrier()
        pltpu.sync_copy(acc_sh.at[pl.ds(sid * RPT, RPT)], wb_v)
        pltpu.sync_copy(wb_v, acc_out.at[cid, pl.ds(sid * RPT, RPT)])

    return pl.kernel(body, out_type=out_type, mesh=mesh,
                     scratch_types=scratch, compiler_params=_SC_PARAMS)


def _counts_body(dst_hbm, z16_hbm, ones_hbm, cnt_out,
                 dst_v, ones_v, wb16_v, cnt_sh):
    cid = lax.axis_index("c")
    sid = lax.axis_index("s")
    wid = sid * NC + cid
    pltpu.sync_copy(z16_hbm, wb16_v)
    pltpu.sync_copy(wb16_v, cnt_sh.at[pl.ds(sid * RPT, RPT)])
    pltpu.sync_copy(ones_hbm, ones_v)
    pltpu.sync_copy(dst_hbm.at[pl.ds(wid * KB, KB)], dst_v)
    plsc.subcore_barrier()

    def step(j, carry):
        pltpu.sync_copy(ones_v, cnt_sh.at[dst_v.at[j]], add=True)
        return carry

    lax.fori_loop(0, KB, step, 0)
    plsc.subcore_barrier()
    pltpu.sync_copy(cnt_sh.at[pl.ds(sid * RPT, RPT)], wb16_v)
    pltpu.sync_copy(wb16_v, cnt_out.at[cid, pl.ds(sid * RPT, RPT)])


_counts = pl.kernel(
    _counts_body,
    out_type=[jax.ShapeDtypeStruct((NC, N_ACC, 16), jnp.float32)],
    mesh=plsc.VectorSubcoreMesh(core_axis_name="c", subcore_axis_name="s"),
    scratch_types=[
        pltpu.VMEM((KB, EB), jnp.int32),
        pltpu.VMEM((EB, 16), jnp.float32),
        pltpu.VMEM((RPT, 16), jnp.float32),
        pltpu.VMEM_SHARED((N_ACC, 16), jnp.float32),
    ],
    compiler_params=_SC_PARAMS,
)


_agg_hid = _make_agg(D_HID)
_agg_out = _make_agg(D_OUT)


def kernel(x, edge_index, W1_l, b1, W1_r, W2_l, b2, W2_r):
    src = edge_index[0].astype(jnp.int32)
    dst = edge_index[1].astype(jnp.int32)
    pad = E_PAD - N_EDGES
    src_p = jnp.concatenate(
        [src, jnp.zeros((pad,), jnp.int32)]).reshape(NW * KB, EB)
    dst_p = jnp.concatenate(
        [dst, jnp.full((pad,), N_NODES, jnp.int32)]).reshape(NW * KB, EB)
    z_hid = jnp.zeros((RPT, D_HID), jnp.float32)
    z_out = jnp.zeros((RPT, D_OUT), jnp.float32)
    z16 = jnp.zeros((RPT, 16), jnp.float32)
    ones = jnp.ones((EB, 16), jnp.float32)

    y1, r1 = _lin1(x, W1_l, W1_r, b1.reshape(1, D_HID))
    (accp,) = _agg_hid(y1, src_p, dst_p, z_hid)
    (cntp,) = _counts(dst_p, z16, ones)
    y2, r2 = _mid(accp[0, :N_NODES], accp[1, :N_NODES],
                  cntp[0, :N_NODES], cntp[1, :N_NODES],
                  r1, W2_l, W2_r, b2.reshape(1, D_OUT))
    (accp2,) = _agg_out(y2, src_p, dst_p, z_out)
    return _out(accp2[0, :N_NODES], accp2[1, :N_NODES],
                cntp[0, :N_NODES], cntp[1, :N_NODES], r2)


# speedup vs baseline: 8.5519x; 1.2467x over previous
import jax
import jax.numpy as jnp
from jax import lax
from jax.experimental import pallas as pl
from jax.experimental.pallas import tpu as pltpu
from jax.experimental.pallas import tpu_sc as plsc

N_NODES = 10000
N_EDGES = 320000
D_IN, D_HID, D_OUT = 128, 64, 32

NC, NS = 2, 16
NW = NC * NS
EB = 128
KB = (-(-N_EDGES // (NW * EB)) + 7) // 8 * 8
E_PAD = NW * KB * EB
RPT = 640
N_ACC = NS * RPT

_ROWS = 1000
_GRID = N_NODES // _ROWS


def _lin1_body(x_ref, wl_ref, wr_ref, b_ref, y_ref, r_ref):
    xb = x_ref[...]
    dn = (((1,), (1,)), ((), ()))
    y_ref[...] = lax.dot_general(xb, wl_ref[...], dn,
                                 preferred_element_type=jnp.float32)
    r_ref[...] = lax.dot_general(xb, wr_ref[...], dn,
                                 preferred_element_type=jnp.float32) + b_ref[...]


_lin1 = pl.pallas_call(
    _lin1_body,
    grid=(_GRID,),
    in_specs=[
        pl.BlockSpec((_ROWS, D_IN), lambda i: (i, 0)),
        pl.BlockSpec((D_HID, D_IN), lambda i: (0, 0)),
        pl.BlockSpec((D_HID, D_IN), lambda i: (0, 0)),
        pl.BlockSpec((1, D_HID), lambda i: (0, 0)),
    ],
    out_specs=[pl.BlockSpec((_ROWS, D_HID), lambda i: (i, 0))] * 2,
    out_shape=[jax.ShapeDtypeStruct((N_NODES, D_HID), jnp.float32)] * 2,
)


def _mid_body(sa_ref, sb_ref, ca_ref, cb_ref, r1_ref, wl_ref, wr_ref, b_ref,
              y_ref, r_ref):
    c = ca_ref[...] + cb_ref[...]
    inv = 1.0 / jnp.maximum(c[:, 0:1], 1.0)
    h = jnp.maximum((sa_ref[...] + sb_ref[...]) * inv + r1_ref[...], 0.0)
    dn = (((1,), (1,)), ((), ()))
    y_ref[...] = lax.dot_general(h, wl_ref[...], dn,
                                 preferred_element_type=jnp.float32)
    r_ref[...] = lax.dot_general(h, wr_ref[...], dn,
                                 preferred_element_type=jnp.float32) + b_ref[...]


_mid = pl.pallas_call(
    _mid_body,
    grid=(_GRID,),
    in_specs=[
        pl.BlockSpec((_ROWS, D_HID), lambda i: (i, 0)),
        pl.BlockSpec((_ROWS, D_HID), lambda i: (i, 0)),
        pl.BlockSpec((_ROWS, 16), lambda i: (i, 0)),
        pl.BlockSpec((_ROWS, 16), lambda i: (i, 0)),
        pl.BlockSpec((_ROWS, D_HID), lambda i: (i, 0)),
        pl.BlockSpec((D_OUT, D_HID), lambda i: (0, 0)),
        pl.BlockSpec((D_OUT, D_HID), lambda i: (0, 0)),
        pl.BlockSpec((1, D_OUT), lambda i: (0, 0)),
    ],
    out_specs=[pl.BlockSpec((_ROWS, D_OUT), lambda i: (i, 0))] * 2,
    out_shape=[jax.ShapeDtypeStruct((N_NODES, D_OUT), jnp.float32)] * 2,
)


def _out_body(sa_ref, sb_ref, ca_ref, cb_ref, r2_ref, o_ref):
    c = ca_ref[...] + cb_ref[...]
    inv = 1.0 / jnp.maximum(c[:, 0:1], 1.0)
    o_ref[...] = (sa_ref[...] + sb_ref[...]) * inv + r2_ref[...]


_out = pl.pallas_call(
    _out_body,
    grid=(_GRID,),
    in_specs=[
        pl.BlockSpec((_ROWS, D_OUT), lambda i: (i, 0)),
        pl.BlockSpec((_ROWS, D_OUT), lambda i: (i, 0)),
        pl.BlockSpec((_ROWS, 16), lambda i: (i, 0)),
        pl.BlockSpec((_ROWS, 16), lambda i: (i, 0)),
        pl.BlockSpec((_ROWS, D_OUT), lambda i: (i, 0)),
    ],
    out_specs=pl.BlockSpec((_ROWS, D_OUT), lambda i: (i, 0)),
    out_shape=jax.ShapeDtypeStruct((N_NODES, D_OUT), jnp.float32),
)


_SC_PARAMS = pltpu.CompilerParams(use_tc_tiling_on_sc=False)


def _make_agg(D):
    mesh = plsc.VectorSubcoreMesh(core_axis_name="c", subcore_axis_name="s")
    out_type = [jax.ShapeDtypeStruct((NC, N_ACC, D), jnp.float32)]
    G = D // 32
    scratch = [
        pltpu.VMEM((KB, EB), jnp.int32),
        pltpu.VMEM((KB, EB), jnp.int32),
        pltpu.VMEM((2 * EB, D // 2), jnp.int32),
        pltpu.VMEM((2 * EB, D), jnp.float32),
        pltpu.VMEM((RPT, D), jnp.float32),
        pltpu.VMEM_SHARED((N_ACC, D), jnp.float32),
        pltpu.SemaphoreType.DMA,
        pltpu.SemaphoreType.DMA,
    ]

    def body(y_hbm, src_hbm, dst_hbm, z_hbm, acc_out,
             src_v, dst_v, rbf_v, rf_v, wb_v, acc_sh, sem_g, sem_s):
        cid = lax.axis_index("c")
        sid = lax.axis_index("s")
        wid = sid * NC + cid
        pltpu.sync_copy(z_hbm, wb_v)
        pltpu.sync_copy(wb_v, acc_sh.at[pl.ds(sid * RPT, RPT)])
        pltpu.sync_copy(src_hbm.at[pl.ds(wid * KB, KB)], src_v)
        pltpu.sync_copy(dst_hbm.at[pl.ds(wid * KB, KB)], dst_v)
        plsc.subcore_barrier()

        pltpu.async_copy(y_hbm.at[src_v.at[0]], rbf_v.at[pl.ds(0, EB)], sem_g)

        def step(j, carry):
            b = lax.rem(j, 2)
            base = b * EB
            pltpu.make_async_copy(
                y_hbm.at[src_v.at[j]], rbf_v.at[pl.ds(base, EB)],
                sem_g).wait()

            @pl.when(j >= 2)
            def _():
                pltpu.make_async_copy(
                    rf_v.at[pl.ds(base, EB)], acc_sh.at[dst_v.at[j]],
                    sem_s).wait()

            @pl.when(j + 1 < KB)
            def _():
                pltpu.async_copy(
                    y_hbm.at[src_v.at[j + 1]],
                    rbf_v.at[pl.ds((EB - base), EB)], sem_g)

            def conv(r, c2):
                row = base + r
                for g in range(G):
                    w = rbf_v[row, pl.ds(16 * g, 16)]
                    rf_v[row, pl.ds(32 * g, 16)] = lax.bitcast_convert_type(
                        w << 16, jnp.float32)
                    rf_v[row, pl.ds(32 * g + 16, 16)] = lax.bitcast_convert_type(
                        w & jnp.int32(-65536), jnp.float32)
                return c2

            lax.fori_loop(0, EB, conv, 0)
            pltpu.async_copy(
                rf_v.at[pl.ds(base, EB)], acc_sh.at[dst_v.at[j]],
                sem_s, add=True)
            return carry

        lax.fori_loop(0, KB, step, 0)
        for j in (KB - 2, KB - 1):
            pltpu.make_async_copy(
                rf_v.at[pl.ds((j % 2) * EB, EB)], acc_sh.at[dst_v.at[j]],
                sem_s).wait()
        plsc.subcore_barrier()
        pltpu.sync_copy(acc_sh.at[pl.ds(sid * RPT, RPT)], wb_v)
        pltpu.sync_copy(wb_v, acc_out.at[cid, pl.ds(sid * RPT, RPT)])

    return pl.kernel(body, out_type=out_type, mesh=mesh,
                     scratch_types=scratch, compiler_params=_SC_PARAMS)


def _counts_body(dst_hbm, z16_hbm, ones_hbm, cnt_out,
                 dst_v, ones_v, wb16_v, cnt_sh):
    cid = lax.axis_index("c")
    sid = lax.axis_index("s")
    wid = sid * NC + cid
    pltpu.sync_copy(z16_hbm, wb16_v)
    pltpu.sync_copy(wb16_v, cnt_sh.at[pl.ds(sid * RPT, RPT)])
    pltpu.sync_copy(ones_hbm, ones_v)
    pltpu.sync_copy(dst_hbm.at[pl.ds(wid * KB, KB)], dst_v)
    plsc.subcore_barrier()

    def step(j, carry):
        pltpu.sync_copy(ones_v, cnt_sh.at[dst_v.at[j]], add=True)
        return carry

    lax.fori_loop(0, KB, step, 0)
    plsc.subcore_barrier()
    pltpu.sync_copy(cnt_sh.at[pl.ds(sid * RPT, RPT)], wb16_v)
    pltpu.sync_copy(wb16_v, cnt_out.at[cid, pl.ds(sid * RPT, RPT)])


_counts = pl.kernel(
    _counts_body,
    out_type=[jax.ShapeDtypeStruct((NC, N_ACC, 16), jnp.float32)],
    mesh=plsc.VectorSubcoreMesh(core_axis_name="c", subcore_axis_name="s"),
    scratch_types=[
        pltpu.VMEM((KB, EB), jnp.int32),
        pltpu.VMEM((EB, 16), jnp.float32),
        pltpu.VMEM((RPT, 16), jnp.float32),
        pltpu.VMEM_SHARED((N_ACC, 16), jnp.float32),
    ],
    compiler_params=_SC_PARAMS,
)


_agg_hid = _make_agg(D_HID)
_agg_out = _make_agg(D_OUT)


def kernel(x, edge_index, W1_l, b1, W1_r, W2_l, b2, W2_r):
    src = edge_index[0].astype(jnp.int32)
    dst = edge_index[1].astype(jnp.int32)
    pad = E_PAD - N_EDGES
    src_p = jnp.concatenate(
        [src, jnp.zeros((pad,), jnp.int32)]).reshape(NW * KB, EB)
    dst_p = jnp.concatenate(
        [dst, jnp.full((pad,), N_NODES, jnp.int32)]).reshape(NW * KB, EB)
    z_hid = jnp.zeros((RPT, D_HID), jnp.float32)
    z_out = jnp.zeros((RPT, D_OUT), jnp.float32)
    z16 = jnp.zeros((RPT, 16), jnp.float32)
    ones = jnp.ones((EB, 16), jnp.float32)

    def bf16_interleaved(y, d):
        yb = (y.astype(jnp.bfloat16)
              .reshape(-1, d // 32, 2, 16).transpose(0, 1, 3, 2)
              .reshape(-1, d // 2, 2))
        return lax.bitcast_convert_type(yb, jnp.int32)

    y1, r1 = _lin1(x, W1_l, W1_r, b1.reshape(1, D_HID))
    (accp,) = _agg_hid(bf16_interleaved(y1, D_HID), src_p, dst_p, z_hid)
    (cntp,) = _counts(dst_p, z16, ones)
    y2, r2 = _mid(accp[0, :N_NODES], accp[1, :N_NODES],
                  cntp[0, :N_NODES], cntp[1, :N_NODES],
                  r1, W2_l, W2_r, b2.reshape(1, D_OUT))
    (accp2,) = _agg_out(bf16_interleaved(y2, D_OUT), src_p, dst_p, z_out)
    return _out(accp2[0, :N_NODES], accp2[1, :N_NODES],
                cntp[0, :N_NODES], cntp[1, :N_NODES], r2)


# speedup vs baseline: 8.9190x; 1.0429x over previous
import jax
import jax.numpy as jnp
from jax import lax
from jax.experimental import pallas as pl
from jax.experimental.pallas import tpu as pltpu
from jax.experimental.pallas import tpu_sc as plsc

N_NODES = 10000
N_EDGES = 320000
D_IN, D_HID, D_OUT = 128, 64, 32

NC, NS = 2, 16
NW = NC * NS
EB = 128
KB = (-(-N_EDGES // (NW * EB)) + 7) // 8 * 8
E_PAD = NW * KB * EB
RPT = 640
N_ACC = NS * RPT

_ROWS = 1000
_GRID = N_NODES // _ROWS


def _lin1_body(x_ref, wl_ref, wr_ref, b_ref, y_ref, r_ref):
    xb = x_ref[...]
    dn = (((1,), (1,)), ((), ()))
    y_ref[...] = lax.dot_general(xb, wl_ref[...], dn,
                                 preferred_element_type=jnp.float32)
    r_ref[...] = lax.dot_general(xb, wr_ref[...], dn,
                                 preferred_element_type=jnp.float32) + b_ref[...]


_lin1 = pl.pallas_call(
    _lin1_body,
    grid=(_GRID,),
    in_specs=[
        pl.BlockSpec((_ROWS, D_IN), lambda i: (i, 0)),
        pl.BlockSpec((D_HID, D_IN), lambda i: (0, 0)),
        pl.BlockSpec((D_HID, D_IN), lambda i: (0, 0)),
        pl.BlockSpec((1, D_HID), lambda i: (0, 0)),
    ],
    out_specs=[pl.BlockSpec((_ROWS, D_HID), lambda i: (i, 0))] * 2,
    out_shape=[jax.ShapeDtypeStruct((N_NODES, D_HID), jnp.float32)] * 2,
)


def _mid_body(sa_ref, sb_ref, ca_ref, cb_ref, r1_ref, wl_ref, wr_ref, b_ref,
              y_ref, r_ref):
    c = ca_ref[...] + cb_ref[...]
    inv = 1.0 / jnp.maximum(c[:, 0:1], 1.0)
    h = jnp.maximum((sa_ref[...] + sb_ref[...]) * inv + r1_ref[...], 0.0)
    dn = (((1,), (1,)), ((), ()))
    y_ref[...] = lax.dot_general(h, wl_ref[...], dn,
                                 preferred_element_type=jnp.float32)
    r_ref[...] = lax.dot_general(h, wr_ref[...], dn,
                                 preferred_element_type=jnp.float32) + b_ref[...]


_mid = pl.pallas_call(
    _mid_body,
    grid=(_GRID,),
    in_specs=[
        pl.BlockSpec((_ROWS, D_HID), lambda i: (i, 0)),
        pl.BlockSpec((_ROWS, D_HID), lambda i: (i, 0)),
        pl.BlockSpec((_ROWS, 16), lambda i: (i, 0)),
        pl.BlockSpec((_ROWS, 16), lambda i: (i, 0)),
        pl.BlockSpec((_ROWS, D_HID), lambda i: (i, 0)),
        pl.BlockSpec((D_OUT, D_HID), lambda i: (0, 0)),
        pl.BlockSpec((D_OUT, D_HID), lambda i: (0, 0)),
        pl.BlockSpec((1, D_OUT), lambda i: (0, 0)),
    ],
    out_specs=[pl.BlockSpec((_ROWS, D_OUT), lambda i: (i, 0))] * 2,
    out_shape=[jax.ShapeDtypeStruct((N_NODES, D_OUT), jnp.float32)] * 2,
)


def _out_body(sa_ref, sb_ref, ca_ref, cb_ref, r2_ref, o_ref):
    c = ca_ref[...] + cb_ref[...]
    inv = 1.0 / jnp.maximum(c[:, 0:1], 1.0)
    o_ref[...] = (sa_ref[...] + sb_ref[...]) * inv + r2_ref[...]


_out = pl.pallas_call(
    _out_body,
    grid=(_GRID,),
    in_specs=[
        pl.BlockSpec((_ROWS, D_OUT), lambda i: (i, 0)),
        pl.BlockSpec((_ROWS, D_OUT), lambda i: (i, 0)),
        pl.BlockSpec((_ROWS, 16), lambda i: (i, 0)),
        pl.BlockSpec((_ROWS, 16), lambda i: (i, 0)),
        pl.BlockSpec((_ROWS, D_OUT), lambda i: (i, 0)),
    ],
    out_specs=pl.BlockSpec((_ROWS, D_OUT), lambda i: (i, 0)),
    out_shape=jax.ShapeDtypeStruct((N_NODES, D_OUT), jnp.float32),
)


_SC_PARAMS = pltpu.CompilerParams(use_tc_tiling_on_sc=False)


def _make_agg(D):
    mesh = plsc.VectorSubcoreMesh(core_axis_name="c", subcore_axis_name="s")
    out_type = [jax.ShapeDtypeStruct((NC, N_ACC, D), jnp.float32)]
    G = D // 32
    scratch = [
        pltpu.VMEM((KB, EB), jnp.int32),
        pltpu.VMEM((KB, EB), jnp.int32),
        pltpu.VMEM((2 * EB, D // 2), jnp.int32),
        pltpu.VMEM((2 * EB, D), jnp.float32),
        pltpu.VMEM_SHARED((N_ACC, D), jnp.float32),
        pltpu.SemaphoreType.DMA,
        pltpu.SemaphoreType.DMA,
    ]

    def body(y_hbm, src_hbm, dst_hbm, z_hbm, acc_out,
             src_v, dst_v, rbf_v, rf_v, acc_sh, sem_g, sem_s):
        cid = lax.axis_index("c")
        sid = lax.axis_index("s")
        wid = sid * NC + cid
        pltpu.sync_copy(z_hbm, acc_sh.at[pl.ds(sid * RPT, RPT)])
        pltpu.sync_copy(src_hbm.at[pl.ds(wid * KB, KB)], src_v)
        pltpu.sync_copy(dst_hbm.at[pl.ds(wid * KB, KB)], dst_v)
        plsc.subcore_barrier()

        pltpu.async_copy(y_hbm.at[src_v.at[0]], rbf_v.at[pl.ds(0, EB)], sem_g)

        def step(j, carry):
            b = lax.rem(j, 2)
            base = b * EB
            pltpu.make_async_copy(
                y_hbm.at[src_v.at[j]], rbf_v.at[pl.ds(base, EB)],
                sem_g).wait()

            @pl.when(j >= 2)
            def _():
                pltpu.make_async_copy(
                    rf_v.at[pl.ds(base, EB)], acc_sh.at[dst_v.at[j]],
                    sem_s).wait()

            @pl.when(j + 1 < KB)
            def _():
                pltpu.async_copy(
                    y_hbm.at[src_v.at[j + 1]],
                    rbf_v.at[pl.ds((EB - base), EB)], sem_g)

            def conv(r, c2):
                row = base + r
                for g in range(G):
                    w = rbf_v[row, pl.ds(16 * g, 16)]
                    rf_v[row, pl.ds(32 * g, 16)] = lax.bitcast_convert_type(
                        w << 16, jnp.float32)
                    rf_v[row, pl.ds(32 * g + 16, 16)] = lax.bitcast_convert_type(
                        w & jnp.int32(-65536), jnp.float32)
                return c2

            lax.fori_loop(0, EB, conv, 0)
            pltpu.async_copy(
                rf_v.at[pl.ds(base, EB)], acc_sh.at[dst_v.at[j]],
                sem_s, add=True)
            return carry

        lax.fori_loop(0, KB, step, 0)
        for j in (KB - 2, KB - 1):
            pltpu.make_async_copy(
                rf_v.at[pl.ds((j % 2) * EB, EB)], acc_sh.at[dst_v.at[j]],
                sem_s).wait()
        plsc.subcore_barrier()
        pltpu.sync_copy(acc_sh.at[pl.ds(sid * RPT, RPT)],
                        acc_out.at[cid, pl.ds(sid * RPT, RPT)])

    return pl.kernel(body, out_type=out_type, mesh=mesh,
                     scratch_types=scratch, compiler_params=_SC_PARAMS)


def _counts_body(dst_hbm, z16_hbm, ones_hbm, cnt_out,
                 dst_v, ones_v, cnt_sh):
    cid = lax.axis_index("c")
    sid = lax.axis_index("s")
    wid = sid * NC + cid
    pltpu.sync_copy(z16_hbm, cnt_sh.at[pl.ds(sid * RPT, RPT)])
    pltpu.sync_copy(ones_hbm, ones_v)
    pltpu.sync_copy(dst_hbm.at[pl.ds(wid * KB, KB)], dst_v)
    plsc.subcore_barrier()

    def step(j, carry):
        pltpu.sync_copy(ones_v, cnt_sh.at[dst_v.at[j]], add=True)
        return carry

    lax.fori_loop(0, KB, step, 0)
    plsc.subcore_barrier()
    pltpu.sync_copy(cnt_sh.at[pl.ds(sid * RPT, RPT)],
                    cnt_out.at[cid, pl.ds(sid * RPT, RPT)])


_counts = pl.kernel(
    _counts_body,
    out_type=[jax.ShapeDtypeStruct((NC, N_ACC, 16), jnp.float32)],
    mesh=plsc.VectorSubcoreMesh(core_axis_name="c", subcore_axis_name="s"),
    scratch_types=[
        pltpu.VMEM((KB, EB), jnp.int32),
        pltpu.VMEM((EB, 16), jnp.float32),
        pltpu.VMEM_SHARED((N_ACC, 16), jnp.float32),
    ],
    compiler_params=_SC_PARAMS,
)


_agg_hid = _make_agg(D_HID)
_agg_out = _make_agg(D_OUT)


def kernel(x, edge_index, W1_l, b1, W1_r, W2_l, b2, W2_r):
    src = edge_index[0].astype(jnp.int32)
    dst = edge_index[1].astype(jnp.int32)
    pad = E_PAD - N_EDGES
    src_p = jnp.concatenate(
        [src, jnp.zeros((pad,), jnp.int32)]).reshape(NW * KB, EB)
    dst_p = jnp.concatenate(
        [dst, jnp.full((pad,), N_NODES, jnp.int32)]).reshape(NW * KB, EB)
    z_hid = jnp.zeros((RPT, D_HID), jnp.float32)
    z_out = jnp.zeros((RPT, D_OUT), jnp.float32)
    z16 = jnp.zeros((RPT, 16), jnp.float32)
    ones = jnp.ones((EB, 16), jnp.float32)

    def bf16_interleaved(y, d):
        yb = (y.astype(jnp.bfloat16)
              .reshape(-1, d // 32, 2, 16).transpose(0, 1, 3, 2)
              .reshape(-1, d // 2, 2))
        return lax.bitcast_convert_type(yb, jnp.int32)

    y1, r1 = _lin1(x, W1_l, W1_r, b1.reshape(1, D_HID))
    (accp,) = _agg_hid(bf16_interleaved(y1, D_HID), src_p, dst_p, z_hid)
    (cntp,) = _counts(dst_p, z16, ones)
    y2, r2 = _mid(accp[0, :N_NODES], accp[1, :N_NODES],
                  cntp[0, :N_NODES], cntp[1, :N_NODES],
                  r1, W2_l, W2_r, b2.reshape(1, D_OUT))
    (accp2,) = _agg_out(bf16_interleaved(y2, D_OUT), src_p, dst_p, z_out)
    return _out(accp2[0, :N_NODES], accp2[1, :N_NODES],
                cntp[0, :N_NODES], cntp[1, :N_NODES], r2)
